# Initial kernel scaffold; baseline (speedup 1.0000x reference)
import jax
import jax.numpy as jnp
from jax.experimental import pallas as pl


def kernel(x, edge_index, batch, edge_attr, params):
    raise NotImplementedError("write your pallas kernel here")


# speedup vs baseline: 2.4343x; 2.4343x over previous
import functools

import jax
import jax.numpy as jnp
from jax import lax
from jax.experimental import pallas as pl
from jax.experimental.pallas import tpu as pltpu
from jax.experimental.pallas import tpu_sc as plsc

N = 10000
E = 320000
F = 128
POS = 3
TBL = 256
MSGW = 128
H1 = 528
CH = 80
BE = 640
BN = 1000


def _silu(v):
    return v * jax.nn.sigmoid(v)


def _sc_gather(table, dst, src):
    info = plsc.get_sparse_core_info()
    nc, ns = info.num_cores, info.num_subcores
    nw = nc * ns
    epw = E // nw
    nch = epw // CH
    mesh = plsc.VectorSubcoreMesh(core_axis_name="c", subcore_axis_name="s")

    @functools.partial(
        pl.kernel,
        mesh=mesh,
        out_type=[jax.ShapeDtypeStruct((E, TBL), jnp.float32),
                  jax.ShapeDtypeStruct((E, TBL), jnp.float32)],
        scratch_types=[pltpu.VMEM((CH,), jnp.int32),
                       pltpu.VMEM((CH,), jnp.int32),
                       pltpu.VMEM((CH, TBL), jnp.float32),
                       pltpu.VMEM((CH, TBL), jnp.float32),
                       pltpu.SemaphoreType.DMA,
                       pltpu.SemaphoreType.DMA],
    )
    def k(tbl_hbm, dst_hbm, src_hbm, gd_hbm, gs_hbm,
          idx_d, idx_s, rows_d, rows_s, sem_d, sem_s):
        wid = lax.axis_index("s") * nc + lax.axis_index("c")
        base = wid * epw

        def body(i, carry):
            off = base + i * CH
            pltpu.sync_copy(dst_hbm.at[pl.ds(off, CH)], idx_d)
            pltpu.sync_copy(src_hbm.at[pl.ds(off, CH)], idx_s)
            cp_d = pltpu.async_copy(tbl_hbm.at[idx_d], rows_d, sem_d)
            cp_s = pltpu.async_copy(tbl_hbm.at[idx_s], rows_s, sem_s)
            cp_d.wait()
            cp_s.wait()
            pltpu.sync_copy(rows_d, gd_hbm.at[pl.ds(off, CH)])
            pltpu.sync_copy(rows_s, gs_hbm.at[pl.ds(off, CH)])
            return carry

        lax.fori_loop(0, nch, body, 0)

    return k(table, dst, src)


def _sc_scatter(msg, dst, zeros_stripe):
    info = plsc.get_sparse_core_info()
    nc, ns = info.num_cores, info.num_subcores
    nw = nc * ns
    epw = E // nw
    nch = epw // CH
    rps = -(-N // ns) // 8 * 8 + 8
    npad = rps * ns
    mesh = plsc.VectorSubcoreMesh(core_axis_name="c", subcore_axis_name="s")

    @functools.partial(
        pl.kernel,
        mesh=mesh,
        out_type=jax.ShapeDtypeStruct((nc, npad, MSGW), jnp.float32),
        scratch_types=[pltpu.VMEM((CH,), jnp.int32),
                       pltpu.VMEM((CH, MSGW), jnp.float32),
                       pltpu.VMEM_SHARED((npad, MSGW), jnp.float32)],
    )
    def k(msg_hbm, dst_hbm, z_hbm, out_hbm, idx_v, rows_v, acc):
        cid = lax.axis_index("c")
        sid = lax.axis_index("s")
        wid = sid * nc + cid
        base = wid * epw
        pltpu.sync_copy(z_hbm, acc.at[pl.ds(sid * rps, rps)])
        plsc.subcore_barrier()

        def body(i, carry):
            off = base + i * CH
            pltpu.sync_copy(dst_hbm.at[pl.ds(off, CH)], idx_v)
            pltpu.sync_copy(msg_hbm.at[pl.ds(off, CH)], rows_v)
            pltpu.sync_copy(rows_v, acc.at[idx_v], add=True)
            return carry

        lax.fori_loop(0, nch, body, 0)
        plsc.subcore_barrier()
        pltpu.sync_copy(acc.at[pl.ds(sid * rps, rps)],
                        out_hbm.at[cid, pl.ds(sid * rps, rps)])

    return k(msg, dst, zeros_stripe)


def _tc_edge(gd, gs, eap, wd, ws, wea, wdr, b1, w2, b2, wc1, bc1, wc2, bc2):
    nb = E // BE

    def body(gd_ref, gs_ref, ea_ref, wd_ref, ws_ref, wea_ref, wdr_ref,
             b1_ref, w2_ref, b2_ref, wc1_ref, bc1_ref, wc2_ref, bc2_ref,
             out_ref):
        g_d = gd_ref[...]
        g_s = gs_ref[...]
        rel = g_s[:, F:F + POS] - g_d[:, F:F + POS]
        rd = jnp.sum(rel * rel, axis=1, keepdims=True)
        h = (jnp.dot(g_d, wd_ref[...], preferred_element_type=jnp.float32)
             + jnp.dot(g_s, ws_ref[...], preferred_element_type=jnp.float32)
             + jnp.dot(ea_ref[...], wea_ref[...],
                       preferred_element_type=jnp.float32)
             + rd * wdr_ref[...]
             + b1_ref[...])
        h = _silu(h)
        m = _silu(jnp.dot(h, w2_ref[...], preferred_element_type=jnp.float32)
                  + b2_ref[...])
        cw = _silu(jnp.dot(m, wc1_ref[...], preferred_element_type=jnp.float32)
                   + bc1_ref[...])
        cw = jnp.dot(cw, wc2_ref[...], preferred_element_type=jnp.float32) \
            + bc2_ref[...]
        out_ref[...] = jnp.concatenate(
            [m, cw * rel, jnp.zeros((BE, MSGW - 19), jnp.float32)], axis=1)

    full = lambda shape: pl.BlockSpec(shape, lambda i: (0,) * len(shape))
    return pl.pallas_call(
        body,
        grid=(nb,),
        in_specs=[
            pl.BlockSpec((BE, TBL), lambda i: (i, 0)),
            pl.BlockSpec((BE, TBL), lambda i: (i, 0)),
            pl.BlockSpec((BE, 8), lambda i: (i, 0)),
            full((TBL, H1)), full((TBL, H1)), full((8, H1)), full((1, H1)),
            full((1, H1)), full((H1, 16)), full((1, 16)),
            full((16, 64)), full((1, 64)), full((64, 1)), full((1, 1)),
        ],
        out_specs=pl.BlockSpec((BE, MSGW), lambda i: (i, 0)),
        out_shape=jax.ShapeDtypeStruct((E, MSGW), jnp.float32),
    )(gd, gs, eap, wd, ws, wea, wdr, b1, w2, b2, wc1, bc1, wc2, bc2)


def _tc_node(table, acc, wn1, bn1, wn2, bn2):
    nb = N // BN

    def body(tbl_ref, acc_ref, wn1_ref, bn1_ref, wn2_ref, bn2_ref, out_ref):
        a = acc_ref[0] + acc_ref[1]
        tbl = tbl_ref[...]
        feats = tbl[:, :F]
        nin = jnp.concatenate([feats, a[:, :16]], axis=1)
        hid = _silu(jnp.dot(nin, wn1_ref[...],
                            preferred_element_type=jnp.float32) + bn1_ref[...])
        hid = jnp.dot(hid, wn2_ref[...],
                      preferred_element_type=jnp.float32) + bn2_ref[...]
        feats_out = feats + hid
        coors_out = tbl[:, F:F + POS] + a[:, 16:16 + POS]
        out_ref[...] = jnp.concatenate(
            [feats_out, coors_out, jnp.zeros((BN, TBL - F - POS), jnp.float32)],
            axis=1)

    full = lambda shape: pl.BlockSpec(shape, lambda i: (0,) * len(shape))
    return pl.pallas_call(
        body,
        grid=(nb,),
        in_specs=[
            pl.BlockSpec((BN, TBL), lambda i: (i, 0)),
            pl.BlockSpec((2, BN, MSGW), lambda i: (0, i, 0)),
            full((F + 16, 2 * F)), full((1, 2 * F)),
            full((2 * F, F)), full((1, F)),
        ],
        out_specs=pl.BlockSpec((BN, TBL), lambda i: (i, 0)),
        out_shape=jax.ShapeDtypeStruct((N, TBL), jnp.float32),
    )(table, acc, wn1, bn1, wn2, bn2)


def _pad_weights(p):
    w1 = jnp.pad(p["We1"], ((0, 0), (0, H1 - p["We1"].shape[1])))
    wd = jnp.pad(w1[:F], ((0, TBL - F), (0, 0)))
    ws = jnp.pad(w1[F:2 * F], ((0, TBL - F), (0, 0)))
    wea = jnp.pad(w1[2 * F:2 * F + 4], ((0, 4), (0, 0)))
    wdr = w1[2 * F + 4:2 * F + 5]
    b1 = jnp.pad(p["be1"], (0, H1 - p["be1"].shape[0])).reshape(1, H1)
    w2 = jnp.pad(p["We2"], ((0, H1 - p["We2"].shape[0]), (0, 0)))
    return dict(wd=wd, ws=ws, wea=wea, wdr=wdr, b1=b1, w2=w2,
                b2=p["be2"].reshape(1, -1),
                wc1=p["Wc1"], bc1=p["bc1"].reshape(1, -1),
                wc2=p["Wc2"], bc2=p["bc2"].reshape(1, -1),
                wn1=p["Wn1"], bn1=p["bn1"].reshape(1, -1),
                wn2=p["Wn2"], bn2=p["bn2"].reshape(1, -1))


def kernel(x, edge_index, batch, edge_attr, params):
    src = edge_index[0]
    dst = edge_index[1]
    table = jnp.concatenate(
        [x[:, POS:], x[:, :POS], jnp.zeros((N, TBL - F - POS), jnp.float32)],
        axis=1)
    eap = jnp.pad(edge_attr, ((0, 0), (0, 4)))
    info = plsc.get_sparse_core_info()
    rps = -(-N // info.num_subcores) // 8 * 8 + 8
    zeros_stripe = jnp.zeros((rps, MSGW), jnp.float32)
    for p in params:
        w = _pad_weights(p)
        gd, gs = _sc_gather(table, dst, src)
        msg = _tc_edge(gd, gs, eap, w["wd"], w["ws"], w["wea"], w["wdr"],
                       w["b1"], w["w2"], w["b2"], w["wc1"], w["bc1"],
                       w["wc2"], w["bc2"])
        acc = _sc_scatter(msg, dst, zeros_stripe)
        table = _tc_node(table, acc, w["wn1"], w["bn1"], w["wn2"], w["bn2"])
    return jnp.concatenate([table[:, F:F + POS], table[:, :F]], axis=1)
